# Initial kernel scaffold; baseline (speedup 1.0000x reference)
import jax
import jax.numpy as jnp
from jax.experimental import pallas as pl


def kernel(ids, edge_index, edge_weights, text_embeddings, W_enc, b_enc, Ws0, Wn0, b0, Ws0r, Wn0r, b0r, Ws1, Wn1, b1, Ws1r, Wn1r, b1r):
    raise NotImplementedError("write your pallas kernel here")


# speedup vs baseline: 3.8384x; 3.8384x over previous
import functools

import jax
import jax.numpy as jnp
from jax import lax
from jax.experimental import pallas as pl
from jax.experimental.pallas import tpu as pltpu
from jax.experimental.pallas import tpu_sc as plsc

N_NODES = 10000
HIDDEN = 128
TEXT_DIM = 256
N_IDS = 512

NC = 2
NS = 16
CHUNK = 128
ROWS_PER_TILE = 624
_ROW_PIECES = [(o, min(CHUNK, ROWS_PER_TILE - o))
               for o in range(0, ROWS_PER_TILE, CHUNK)]
_TAIL_BASE = NS * ROWS_PER_TILE
_TAIL_ROWS = N_NODES - _TAIL_BASE

_MESH = plsc.VectorSubcoreMesh(core_axis_name="c", subcore_axis_name="s",
                               num_cores=NC, num_subcores=NS)
_F32 = jnp.float32


def _tile_row_copies(sid, mk_copy):
    rbase = sid * ROWS_PER_TILE
    for off, sz in _ROW_PIECES:
        mk_copy(rbase + off, sz)

    @pl.when(sid == NS - 1)
    def _():
        mk_copy(_TAIL_BASE, _TAIL_ROWS)


def _zero_rows_buf(rows):
    def zrow(r, _):
        for j in range(HIDDEN // 16):
            rows[r, pl.ds(16 * j, 16)] = jnp.zeros((16,), _F32)
        return 0
    lax.fori_loop(0, CHUNK, zrow, 0)


def _make_spmm(e_pad):
    ept = e_pad // NS
    n_chunks = ept // CHUNK

    @functools.partial(
        pl.kernel,
        out_type=jax.ShapeDtypeStruct((NC, N_NODES, HIDDEN), _F32),
        mesh=_MESH,
        scratch_types=[
            pltpu.VMEM((CHUNK,), jnp.int32),
            pltpu.VMEM((CHUNK,), jnp.int32),
            pltpu.VMEM((CHUNK,), jnp.int32),
            pltpu.VMEM((CHUNK,), jnp.int32),
            pltpu.VMEM((CHUNK,), _F32),
            pltpu.VMEM((CHUNK, HIDDEN), _F32),
            pltpu.VMEM_SHARED((N_NODES, HIDDEN), _F32),
            pltpu.SemaphoreType.DMA,
        ],
    )
    def spmm(x_hbm, src_hbm, dst_hbm, ew_hbm, agg_hbm,
             srcv, dstv, idxg, idxs, ewv, rows, acc, sem):
        cid = lax.axis_index("c")
        sid = lax.axis_index("s")
        fwd = cid == 0

        _zero_rows_buf(rows)
        _tile_row_copies(sid, lambda b, s: pltpu.sync_copy(
            rows.at[pl.ds(0, s)], acc.at[pl.ds(b, s)]))
        plsc.subcore_barrier()

        ebase = sid * ept

        def chunk(c, _):
            o = ebase + c * CHUNK
            pltpu.sync_copy(src_hbm.at[pl.ds(o, CHUNK)], srcv)
            pltpu.sync_copy(dst_hbm.at[pl.ds(o, CHUNK)], dstv)
            pltpu.sync_copy(ew_hbm.at[pl.ds(o, CHUNK)], ewv)

            def pick(g, _):
                sl = pl.ds(g * 16, 16)
                sv, dv = srcv[sl], dstv[sl]
                idxg[sl] = jnp.where(fwd, sv, dv)
                idxs[sl] = jnp.where(fwd, dv, sv)
                return 0
            lax.fori_loop(0, CHUNK // 16, pick, 0)

            pltpu.async_copy(x_hbm.at[idxg], rows, sem).wait()

            def scale(g, _):
                ev = ewv[pl.ds(g * 16, 16)]
                for j in range(16):
                    e = g * 16 + j
                    s = ev[j]
                    for k in range(HIDDEN // 16):
                        rows[e, pl.ds(16 * k, 16)] = (
                            rows[e, pl.ds(16 * k, 16)] * s)
                return 0
            lax.fori_loop(0, CHUNK // 16, scale, 0)

            pltpu.sync_copy(rows, acc.at[idxs], add=True)
            return 0

        lax.fori_loop(0, n_chunks, chunk, 0)
        plsc.subcore_barrier()

        _tile_row_copies(sid, lambda b, s: pltpu.sync_copy(
            acc.at[pl.ds(b, s)], agg_hbm.at[cid, pl.ds(b, s)]))

    return spmm


def _make_deg(e_pad):
    ept = e_pad // NS
    n_chunks = ept // CHUNK

    @functools.partial(
        pl.kernel,
        out_type=jax.ShapeDtypeStruct((NC, N_NODES, HIDDEN), _F32),
        mesh=_MESH,
        scratch_types=[
            pltpu.VMEM((CHUNK,), jnp.int32),
            pltpu.VMEM((CHUNK,), jnp.int32),
            pltpu.VMEM((CHUNK,), jnp.int32),
            pltpu.VMEM((CHUNK, HIDDEN), _F32),
            pltpu.VMEM_SHARED((N_NODES + 8, HIDDEN), _F32),
        ],
    )
    def deg(src_hbm, dst_hbm, deg_hbm, srcv, dstv, idxv, buf, acc):
        cid = lax.axis_index("c")
        sid = lax.axis_index("s")
        fwd = cid == 0

        _zero_rows_buf(buf)
        _tile_row_copies(sid, lambda b, s: pltpu.sync_copy(
            buf.at[pl.ds(0, s)], acc.at[pl.ds(b, s)]))

        @pl.when(sid == NS - 1)
        def _():
            pltpu.sync_copy(buf.at[pl.ds(0, 8)], acc.at[pl.ds(N_NODES, 8)])

        def orow(r, _):
            for j in range(HIDDEN // 16):
                buf[r, pl.ds(16 * j, 16)] = jnp.full((16,), 1.0, _F32)
            return 0
        lax.fori_loop(0, CHUNK, orow, 0)
        plsc.subcore_barrier()

        ebase = sid * ept

        def chunk(c, _):
            o = ebase + c * CHUNK
            pltpu.sync_copy(src_hbm.at[pl.ds(o, CHUNK)], srcv)
            pltpu.sync_copy(dst_hbm.at[pl.ds(o, CHUNK)], dstv)

            def pick(g, _):
                sl = pl.ds(g * 16, 16)
                idxv[sl] = jnp.where(fwd, dstv[sl], srcv[sl])
                return 0
            lax.fori_loop(0, CHUNK // 16, pick, 0)

            pltpu.sync_copy(buf, acc.at[idxv], add=True)
            return 0

        lax.fori_loop(0, n_chunks, chunk, 0)
        plsc.subcore_barrier()

        _tile_row_copies(sid, lambda b, s: pltpu.sync_copy(
            acc.at[pl.ds(b, s)], deg_hbm.at[cid, pl.ds(b, s)]))

    return deg


@functools.partial(
    pl.kernel,
    out_type=jax.ShapeDtypeStruct((N_IDS, HIDDEN), _F32),
    mesh=_MESH,
    scratch_types=[
        pltpu.VMEM((N_IDS // (NC * NS),), jnp.int32),
        pltpu.VMEM((N_IDS // (NC * NS), HIDDEN), _F32),
        pltpu.SemaphoreType.DMA,
    ],
)
def _sel(x_hbm, ids_hbm, out_hbm, idxv, rows, sem):
    per = N_IDS // (NC * NS)
    wid = lax.axis_index("s") * NC + lax.axis_index("c")
    base = wid * per
    pltpu.sync_copy(ids_hbm.at[pl.ds(base, per)], idxv)
    pltpu.async_copy(x_hbm.at[idxv], rows, sem).wait()
    pltpu.sync_copy(rows, out_hbm.at[pl.ds(base, per)])


_ROWS_BLK = 2000
_N_BLKS = N_NODES // _ROWS_BLK
_HIGH = jax.lax.Precision.HIGHEST


def _enc_body(t_ref, w_ref, b_ref, o_ref):
    o_ref[...] = (jnp.dot(t_ref[...], w_ref[...],
                          preferred_element_type=_F32, precision=_HIGH)
                  + b_ref[...])


def _enc(text, W, b2d):
    return pl.pallas_call(
        _enc_body,
        grid=(_N_BLKS,),
        in_specs=[
            pl.BlockSpec((_ROWS_BLK, TEXT_DIM), lambda i: (i, 0)),
            pl.BlockSpec((TEXT_DIM, HIDDEN), lambda i: (0, 0)),
            pl.BlockSpec((1, HIDDEN), lambda i: (0, 0)),
        ],
        out_specs=pl.BlockSpec((_ROWS_BLK, HIDDEN), lambda i: (i, 0)),
        out_shape=jax.ShapeDtypeStruct((N_NODES, HIDDEN), _F32),
    )(text, W, b2d)


def _combine_body(x_ref, af_ref, ar_ref, df_ref, dr_ref,
                  ws_ref, wn_ref, b_ref, wsr_ref, wnr_ref, br_ref, o_ref):
    x = x_ref[...]
    nf = af_ref[...] / jnp.maximum(df_ref[...], 1.0)
    nr = ar_ref[...] / jnp.maximum(dr_ref[...], 1.0)
    yf = (jnp.dot(x, ws_ref[...], preferred_element_type=_F32, precision=_HIGH)
          + jnp.dot(nf, wn_ref[...], preferred_element_type=_F32,
                    precision=_HIGH) + b_ref[...])
    yr = (jnp.dot(x, wsr_ref[...], preferred_element_type=_F32,
                  precision=_HIGH)
          + jnp.dot(nr, wnr_ref[...], preferred_element_type=_F32,
                    precision=_HIGH) + br_ref[...])
    o_ref[...] = x + jnp.maximum(yf, 0.0) + jnp.maximum(yr, 0.0)


def _combine(x, aggf, aggr, degf, degr, Ws, Wn, b2d, Wsr, Wnr, br2d):
    blk = lambda r, c: pl.BlockSpec((r, c), lambda i: (i, 0))
    fixed = lambda r, c: pl.BlockSpec((r, c), lambda i: (0, 0))
    return pl.pallas_call(
        _combine_body,
        grid=(_N_BLKS,),
        in_specs=[
            blk(_ROWS_BLK, HIDDEN), blk(_ROWS_BLK, HIDDEN),
            blk(_ROWS_BLK, HIDDEN), blk(_ROWS_BLK, 1), blk(_ROWS_BLK, 1),
            fixed(HIDDEN, HIDDEN), fixed(HIDDEN, HIDDEN), fixed(1, HIDDEN),
            fixed(HIDDEN, HIDDEN), fixed(HIDDEN, HIDDEN), fixed(1, HIDDEN),
        ],
        out_specs=pl.BlockSpec((_ROWS_BLK, HIDDEN), lambda i: (i, 0)),
        out_shape=jax.ShapeDtypeStruct((N_NODES, HIDDEN), _F32),
    )(x, aggf, aggr, degf, degr, Ws, Wn, b2d, Wsr, Wnr, br2d)


def _norm_body(f_ref, o_ref):
    f = f_ref[...]
    o_ref[...] = f / jnp.sqrt(jnp.sum(f * f, axis=1, keepdims=True))


def _norm(feats):
    return pl.pallas_call(
        _norm_body,
        out_shape=jax.ShapeDtypeStruct((N_IDS, HIDDEN), _F32),
    )(feats)


_E_PAD_MULT = NS * CHUNK


def kernel(ids, edge_index, edge_weights, text_embeddings, W_enc, b_enc,
           Ws0, Wn0, b0, Ws0r, Wn0r, b0r,
           Ws1, Wn1, b1, Ws1r, Wn1r, b1r):
    e = edge_weights.shape[0]
    e_pad = -(-e // _E_PAD_MULT) * _E_PAD_MULT
    pad = e_pad - e

    src = edge_index[0].astype(jnp.int32)
    dst = edge_index[1].astype(jnp.int32)
    srcp = jnp.concatenate([src, jnp.zeros((pad,), jnp.int32)])
    dstp = jnp.concatenate([dst, jnp.zeros((pad,), jnp.int32)])
    ewp = jnp.concatenate([edge_weights.astype(_F32), jnp.zeros((pad,), _F32)])
    trash = jnp.full((pad,), N_NODES, jnp.int32)
    srcp2 = jnp.concatenate([src, trash])
    dstp2 = jnp.concatenate([dst, trash])

    spmm = _make_spmm(e_pad)
    degk = _make_deg(e_pad)

    deg2 = degk(srcp2, dstp2)
    degf, degr = deg2[0, :, :1], deg2[1, :, :1]

    x = _enc(text_embeddings.astype(_F32), W_enc, b_enc.reshape(1, -1))

    agg2 = spmm(x, srcp, dstp, ewp)
    x = _combine(x, agg2[0], agg2[1], degf, degr,
                 Ws0, Wn0, b0.reshape(1, -1), Ws0r, Wn0r, b0r.reshape(1, -1))

    agg2 = spmm(x, srcp, dstp, ewp)
    x = _combine(x, agg2[0], agg2[1], degf, degr,
                 Ws1, Wn1, b1.reshape(1, -1), Ws1r, Wn1r, b1r.reshape(1, -1))

    feats = _sel(x, ids.astype(jnp.int32))
    return _norm(feats)
